# baseline (speedup 1.0000x reference)
import functools

import jax
import jax.numpy as jnp
from jax import lax
from jax.experimental import pallas as pl
from jax.experimental.pallas import tpu as pltpu
from jax.experimental.pallas import tpu_sc as plsc

_SC_CH = 16384
_SC_UNIT = 524288
_SC_UNITS = 2

_TC_CHUNK = 524288
_TC_SUB = 16384
_TC_ROWS = _TC_SUB // 128

_LP = (3.51102136e-06, 0.999792362, -0.496977431, 0.314589174,
       -0.188780824, 0.0817256453, -0.0172077992)


def _log1p_poly(e):
    acc = jnp.full_like(e, _LP[6])
    for coef in _LP[5::-1]:
        acc = acc * e + coef
    return acc


def _masked_bce(x, t, log1p_fn):
    tf = t.astype(jnp.float32)
    zf = jnp.maximum(tf, 0.0)
    mf = jnp.minimum(tf + 1.0, 1.0)
    e = jnp.exp(-jnp.abs(x))
    sp = jnp.maximum(x, 0.0) + log1p_fn(e)
    return mf * sp - x * zf, mf


def _make_sc_loss(n_skip, n_sc):
    info = plsc.get_sparse_core_info()
    nc, ns = info.num_cores, info.num_subcores
    nw = nc * ns
    per_w = n_sc // nw
    n_ch = per_w // _SC_CH

    mesh = plsc.VectorSubcoreMesh(core_axis_name="c", subcore_axis_name="s")

    @functools.partial(
        pl.kernel, mesh=mesh,
        out_type=jax.ShapeDtypeStruct((2, nw, 16), jnp.float32),
        scratch_types=[
            pltpu.VMEM((2, _SC_CH), jnp.float32),
            pltpu.VMEM((2, _SC_CH), jnp.int32),
            pltpu.VMEM((2, 16), jnp.float32),
            pltpu.SemaphoreType.DMA,
            pltpu.SemaphoreType.DMA,
            pltpu.SemaphoreType.DMA,
            pltpu.SemaphoreType.DMA,
        ],
    )
    def sc_loss(x_hbm, t_hbm, out_hbm, x_v, t_v, part_v, sx0, sx1, st0, st1):
        wid = lax.axis_index("s") * nc + lax.axis_index("c")
        base = n_skip + wid * per_w
        sx = (sx0, sx1)
        st = (st0, st1)

        def start(ci):
            b = ci % 2
            span = pl.ds(base + ci * _SC_CH, _SC_CH)
            hx = pltpu.async_copy(x_hbm.at[span], x_v.at[b], sx[b])
            ht = pltpu.async_copy(t_hbm.at[span], t_v.at[b], st[b])
            return hx, ht

        handles = {0: start(0)}
        z = jnp.zeros((16,), jnp.float32)
        carry = (z, z, z, z)
        for ci in range(n_ch):
            if ci + 1 < n_ch:
                handles[ci + 1] = start(ci + 1)
            hx, ht = handles.pop(ci)
            hx.wait()
            ht.wait()
            b = ci % 2
            xb = x_v.at[b]
            tb = t_v.at[b]

            def vec_body(j, sc4, xb=xb, tb=tb):
                sa, ca, sb, cb = sc4
                x0 = xb[pl.ds(j * 32, 16)]
                t0 = tb[pl.ds(j * 32, 16)]
                x1 = xb[pl.ds(j * 32 + 16, 16)]
                t1 = tb[pl.ds(j * 32 + 16, 16)]
                ds0, dc0 = _masked_bce(x0, t0, _log1p_poly)
                ds1, dc1 = _masked_bce(x1, t1, _log1p_poly)
                return sa + ds0, ca + dc0, sb + ds1, cb + dc1

            carry = lax.fori_loop(0, _SC_CH // 32, vec_body, carry, unroll=4)

        sa, ca, sb, cb = carry
        part_v[0] = sa + sb
        part_v[1] = ca + cb
        pltpu.sync_copy(part_v.at[0], out_hbm.at[0, wid])
        pltpu.sync_copy(part_v.at[1], out_hbm.at[1, wid])

    return sc_loss


def _tc_body(x_ref, t_ref, out_ref, acc_ref):
    i = pl.program_id(0)

    @pl.when(i == 0)
    def _init():
        acc_ref[...] = jnp.zeros_like(acc_ref)

    def step(j, carry):
        s, c = carry
        x = x_ref[pl.ds(j * _TC_SUB, _TC_SUB)].reshape(_TC_ROWS, 128)
        t = t_ref[pl.ds(j * _TC_SUB, _TC_SUB)].reshape(_TC_ROWS, 128)
        ds, dc = _masked_bce(x, t, jnp.log1p)
        return s + ds, c + dc

    init = (jnp.zeros((_TC_ROWS, 128), jnp.float32),
            jnp.zeros((_TC_ROWS, 128), jnp.float32))
    s, c = jax.lax.fori_loop(0, _TC_CHUNK // _TC_SUB, step, init, unroll=4)
    acc_ref[0] += s
    acc_ref[1] += c

    @pl.when(i == pl.num_programs(0) - 1)
    def _fin():
        out_ref[0] = jnp.sum(acc_ref[0])
        out_ref[1] = jnp.sum(acc_ref[1])


def _tc_loss(output, target, n_tc):
    grid = n_tc // _TC_CHUNK
    return pl.pallas_call(
        _tc_body,
        grid=(grid,),
        in_specs=[
            pl.BlockSpec((_TC_CHUNK,), lambda i: (i,)),
            pl.BlockSpec((_TC_CHUNK,), lambda i: (i,)),
        ],
        out_specs=pl.BlockSpec(memory_space=pltpu.SMEM),
        out_shape=jax.ShapeDtypeStruct((2,), jnp.float32),
        scratch_shapes=[pltpu.VMEM((2, _TC_ROWS, 128), jnp.float32)],
    )(output, target)


def kernel(output, target):
    n = output.shape[0]
    n_sc = _SC_UNITS * _SC_UNIT
    n_tc = n - n_sc
    sc_parts = _make_sc_loss(n_tc, n_sc)(output, target)
    tc_parts = _tc_loss(output, target, n_tc)
    s = tc_parts[0] + jnp.sum(sc_parts[0])
    c = tc_parts[1] + jnp.sum(sc_parts[1])
    return s / c


# speedup vs baseline: 1.4136x; 1.4136x over previous
import functools

import jax
import jax.numpy as jnp
from jax import lax
from jax.experimental import pallas as pl
from jax.experimental.pallas import tpu as pltpu
from jax.experimental.pallas import tpu_sc as plsc

_SC_CH = 16384
_SC_UNIT = 524288
_SC_UNITS = 0

_TC_CHUNK = 524288
_TC_SUB = 16384
_TC_ROWS = _TC_SUB // 128

_LP = (3.51102136e-06, 0.999792362, -0.496977431, 0.314589174,
       -0.188780824, 0.0817256453, -0.0172077992)


def _log1p_poly(e):
    acc = jnp.full_like(e, _LP[6])
    for coef in _LP[5::-1]:
        acc = acc * e + coef
    return acc


def _masked_bce(x, t, log1p_fn):
    tf = t.astype(jnp.float32)
    zf = jnp.maximum(tf, 0.0)
    mf = jnp.minimum(tf + 1.0, 1.0)
    e = jnp.exp(-jnp.abs(x))
    sp = jnp.maximum(x, 0.0) + log1p_fn(e)
    return mf * sp - x * zf, mf


def _make_sc_loss(n_skip, n_sc):
    info = plsc.get_sparse_core_info()
    nc, ns = info.num_cores, info.num_subcores
    nw = nc * ns
    per_w = n_sc // nw
    n_ch = per_w // _SC_CH

    mesh = plsc.VectorSubcoreMesh(core_axis_name="c", subcore_axis_name="s")

    @functools.partial(
        pl.kernel, mesh=mesh,
        out_type=jax.ShapeDtypeStruct((2, nw, 16), jnp.float32),
        scratch_types=[
            pltpu.VMEM((2, _SC_CH), jnp.float32),
            pltpu.VMEM((2, _SC_CH), jnp.int32),
            pltpu.VMEM((2, 16), jnp.float32),
            pltpu.SemaphoreType.DMA,
            pltpu.SemaphoreType.DMA,
            pltpu.SemaphoreType.DMA,
            pltpu.SemaphoreType.DMA,
        ],
    )
    def sc_loss(x_hbm, t_hbm, out_hbm, x_v, t_v, part_v, sx0, sx1, st0, st1):
        wid = lax.axis_index("s") * nc + lax.axis_index("c")
        base = n_skip + wid * per_w
        sx = (sx0, sx1)
        st = (st0, st1)

        def start(ci):
            b = ci % 2
            span = pl.ds(base + ci * _SC_CH, _SC_CH)
            hx = pltpu.async_copy(x_hbm.at[span], x_v.at[b], sx[b])
            ht = pltpu.async_copy(t_hbm.at[span], t_v.at[b], st[b])
            return hx, ht

        handles = {0: start(0)}
        z = jnp.zeros((16,), jnp.float32)
        carry = (z, z, z, z)
        for ci in range(n_ch):
            if ci + 1 < n_ch:
                handles[ci + 1] = start(ci + 1)
            hx, ht = handles.pop(ci)
            hx.wait()
            ht.wait()
            b = ci % 2
            xb = x_v.at[b]
            tb = t_v.at[b]

            def vec_body(j, sc4, xb=xb, tb=tb):
                sa, ca, sb, cb = sc4
                x0 = xb[pl.ds(j * 32, 16)]
                t0 = tb[pl.ds(j * 32, 16)]
                x1 = xb[pl.ds(j * 32 + 16, 16)]
                t1 = tb[pl.ds(j * 32 + 16, 16)]
                ds0, dc0 = _masked_bce(x0, t0, _log1p_poly)
                ds1, dc1 = _masked_bce(x1, t1, _log1p_poly)
                return sa + ds0, ca + dc0, sb + ds1, cb + dc1

            carry = lax.fori_loop(0, _SC_CH // 32, vec_body, carry, unroll=4)

        sa, ca, sb, cb = carry
        part_v[0] = sa + sb
        part_v[1] = ca + cb
        pltpu.sync_copy(part_v.at[0], out_hbm.at[0, wid])
        pltpu.sync_copy(part_v.at[1], out_hbm.at[1, wid])

    return sc_loss


def _tc_body(x_ref, t_ref, out_ref, acc_ref):
    i = pl.program_id(0)

    @pl.when(i == 0)
    def _init():
        acc_ref[...] = jnp.zeros_like(acc_ref)

    def step(j, carry):
        s, c = carry
        x = x_ref[pl.ds(j * _TC_SUB, _TC_SUB)].reshape(_TC_ROWS, 128)
        t = t_ref[pl.ds(j * _TC_SUB, _TC_SUB)].reshape(_TC_ROWS, 128)
        ds, dc = _masked_bce(x, t, jnp.log1p)
        return s + ds, c + dc

    init = (jnp.zeros((_TC_ROWS, 128), jnp.float32),
            jnp.zeros((_TC_ROWS, 128), jnp.float32))
    s, c = jax.lax.fori_loop(0, _TC_CHUNK // _TC_SUB, step, init, unroll=4)
    acc_ref[0] += s
    acc_ref[1] += c

    @pl.when(i == pl.num_programs(0) - 1)
    def _fin():
        out_ref[0] = jnp.sum(acc_ref[0])
        out_ref[1] = jnp.sum(acc_ref[1])


def _tc_loss(output, target, n_tc):
    grid = n_tc // _TC_CHUNK
    return pl.pallas_call(
        _tc_body,
        grid=(grid,),
        in_specs=[
            pl.BlockSpec((_TC_CHUNK,), lambda i: (i,)),
            pl.BlockSpec((_TC_CHUNK,), lambda i: (i,)),
        ],
        out_specs=pl.BlockSpec(memory_space=pltpu.SMEM),
        out_shape=jax.ShapeDtypeStruct((2,), jnp.float32),
        scratch_shapes=[pltpu.VMEM((2, _TC_ROWS, 128), jnp.float32)],
    )(output, target)


def kernel(output, target):
    n = output.shape[0]
    n_sc = _SC_UNITS * _SC_UNIT
    n_tc = n - n_sc
    tc_parts = _tc_loss(output, target, n_tc)
    if n_sc:
        sc_parts = _make_sc_loss(n_tc, n_sc)(output, target)
        s = tc_parts[0] + jnp.sum(sc_parts[0])
        c = tc_parts[1] + jnp.sum(sc_parts[1])
    else:
        s, c = tc_parts[0], tc_parts[1]
    return s / c
